# Initial kernel scaffold; baseline (speedup 1.0000x reference)
import jax
import jax.numpy as jnp
from jax.experimental import pallas as pl


def kernel(x, codebooks):
    raise NotImplementedError("write your pallas kernel here")


# speedup vs baseline: 1.2782x; 1.2782x over previous
import functools

import jax
import jax.numpy as jnp
from jax.experimental import pallas as pl
from jax.experimental.pallas import tpu as pltpu

_B, _S, _D = 4, 4096, 128
_Q, _K = 4, 512
_W = 0.25
_N = _B * _S
_TB = 2048
_GRID = _N // _TB
_LOSS_SCALE = _W / (_B * _S * _D)


def _rvq_block(x_ref, cb_ref, q_ref, idx_ref, loss_ref):
    step = pl.program_id(0)
    x = x_ref[...]
    residual = x
    quantized = jnp.zeros_like(x)
    loss = jnp.zeros((), jnp.float32)
    iota_k = jax.lax.broadcasted_iota(jnp.int32, (_TB, _K), 1)
    idx_cols = []
    for i in range(_Q):
        cb = cb_ref[i]
        c2 = jnp.sum(cb * cb, axis=-1)
        cross = jax.lax.dot_general(
            residual, cb, (((1,), (1,)), ((), ())),
            preferred_element_type=jnp.float32)
        r2 = jnp.sum(residual * residual, axis=-1, keepdims=True)
        d2 = r2 - 2.0 * cross + c2[None, :]
        mn = jnp.min(d2, axis=-1, keepdims=True)
        idx = jnp.min(jnp.where(d2 == mn, iota_k, _K), axis=-1,
                      keepdims=True)
        idx_cols.append(idx)
        onehot = (iota_k == idx).astype(jnp.float32)
        q_step = jnp.dot(onehot, cb,
                         precision=jax.lax.Precision.HIGHEST,
                         preferred_element_type=jnp.float32)
        residual = residual - q_step
        quantized = quantized + q_step
        loss = loss + jnp.sum(residual * residual)
    q_ref[...] = quantized
    idx_ref[...] = jnp.concatenate(idx_cols, axis=1)

    @pl.when(step == 0)
    def _init():
        loss_ref[...] = jnp.zeros((1, 1), jnp.float32)

    loss_ref[...] += (loss * _LOSS_SCALE).reshape(1, 1)


@jax.jit
def kernel(x, codebooks):
    xf = x.reshape(_N, _D)
    quantized, indices, loss = pl.pallas_call(
        _rvq_block,
        grid=(_GRID,),
        in_specs=[
            pl.BlockSpec((_TB, _D), lambda i: (i, 0)),
            pl.BlockSpec((_Q, _K, _D), lambda i: (0, 0, 0)),
        ],
        out_specs=[
            pl.BlockSpec((_TB, _D), lambda i: (i, 0)),
            pl.BlockSpec((_TB, _Q), lambda i: (i, 0)),
            pl.BlockSpec((1, 1), lambda i: (0, 0)),
        ],
        out_shape=[
            jax.ShapeDtypeStruct((_N, _D), jnp.float32),
            jax.ShapeDtypeStruct((_N, _Q), jnp.int32),
            jax.ShapeDtypeStruct((1, 1), jnp.float32),
        ],
        compiler_params=pltpu.CompilerParams(
            dimension_semantics=("arbitrary",),
        ),
    )(xf, codebooks)
    return (quantized.reshape(_B, _S, _D),
            indices.reshape(_B, _S, _Q),
            loss.reshape(()))


# speedup vs baseline: 2.0050x; 1.5685x over previous
import functools

import jax
import jax.numpy as jnp
from jax.experimental import pallas as pl
from jax.experimental.pallas import tpu as pltpu

_B, _S, _D = 4, 4096, 128
_Q, _K = 4, 512
_W = 0.25
_N = _B * _S
_TB = 2048
_GRID = _N // _TB
_LOSS_SCALE = _W / (_B * _S * _D)


def _rvq_block(x_ref, cb_ref, q_ref, idx_ref, loss_ref):
    step = pl.program_id(0)
    x = x_ref[...]
    residual = x
    quantized = jnp.zeros_like(x)
    loss = jnp.zeros((), jnp.float32)
    iota_k = jax.lax.broadcasted_iota(jnp.int32, (_TB, _K), 1)
    idx_cols = []
    for i in range(_Q):
        cb = cb_ref[i]
        c2 = jnp.sum(cb * cb, axis=-1)
        cross = jax.lax.dot_general(
            residual, cb, (((1,), (1,)), ((), ())),
            preferred_element_type=jnp.float32)
        r2 = jnp.sum(residual * residual, axis=-1, keepdims=True)
        d2 = r2 - 2.0 * cross + c2[None, :]
        idx = jnp.argmin(d2, axis=-1).reshape(_TB, 1)
        idx_cols.append(idx)
        onehot = (iota_k == idx).astype(jnp.bfloat16)
        cb_hi = cb.astype(jnp.bfloat16)
        rem1 = cb - cb_hi.astype(jnp.float32)
        cb_mid = rem1.astype(jnp.bfloat16)
        cb_lo = (rem1 - cb_mid.astype(jnp.float32)).astype(jnp.bfloat16)
        q_step = jnp.dot(onehot, cb_hi, preferred_element_type=jnp.float32)
        q_step = q_step + jnp.dot(onehot, cb_mid,
                                  preferred_element_type=jnp.float32)
        q_step = q_step + jnp.dot(onehot, cb_lo,
                                  preferred_element_type=jnp.float32)
        residual = residual - q_step
        quantized = quantized + q_step
        loss = loss + jnp.sum(residual * residual)
    q_ref[...] = quantized
    idx_ref[...] = jnp.concatenate(idx_cols, axis=1)

    @pl.when(step == 0)
    def _init():
        loss_ref[...] = jnp.zeros((1, 1), jnp.float32)

    loss_ref[...] += (loss * _LOSS_SCALE).reshape(1, 1)


@jax.jit
def kernel(x, codebooks):
    xf = x.reshape(_N, _D)
    quantized, indices, loss = pl.pallas_call(
        _rvq_block,
        grid=(_GRID,),
        in_specs=[
            pl.BlockSpec((_TB, _D), lambda i: (i, 0)),
            pl.BlockSpec((_Q, _K, _D), lambda i: (0, 0, 0)),
        ],
        out_specs=[
            pl.BlockSpec((_TB, _D), lambda i: (i, 0)),
            pl.BlockSpec((_TB, _Q), lambda i: (i, 0)),
            pl.BlockSpec((1, 1), lambda i: (0, 0)),
        ],
        out_shape=[
            jax.ShapeDtypeStruct((_N, _D), jnp.float32),
            jax.ShapeDtypeStruct((_N, _Q), jnp.int32),
            jax.ShapeDtypeStruct((1, 1), jnp.float32),
        ],
        compiler_params=pltpu.CompilerParams(
            dimension_semantics=("arbitrary",),
        ),
    )(xf, codebooks)
    return (quantized.reshape(_B, _S, _D),
            indices.reshape(_B, _S, _Q),
            loss.reshape(()))


# speedup vs baseline: 2.1380x; 1.0664x over previous
import functools

import jax
import jax.numpy as jnp
from jax.experimental import pallas as pl
from jax.experimental.pallas import tpu as pltpu

_B, _S, _D = 4, 4096, 128
_Q, _K = 4, 512
_W = 0.25
_N = _B * _S
_TB = 4096
_GRID = _N // _TB
_LOSS_SCALE = _W / (_B * _S * _D)


def _rvq_block(x_ref, cb_ref, q_ref, idx_ref, loss_ref):
    step = pl.program_id(0)
    x = x_ref[...]
    residual = x
    quantized = jnp.zeros_like(x)
    loss = jnp.zeros((), jnp.float32)
    iota_k = jax.lax.broadcasted_iota(jnp.int32, (_TB, _K), 1)
    idx_cols = []
    for i in range(_Q):
        cb = cb_ref[i]
        c2 = jnp.sum(cb * cb, axis=-1)
        cross = jax.lax.dot_general(
            residual, cb, (((1,), (1,)), ((), ())),
            preferred_element_type=jnp.float32)
        r2 = jnp.sum(residual * residual, axis=-1, keepdims=True)
        d2 = r2 - 2.0 * cross + c2[None, :]
        idx = jnp.argmin(d2, axis=-1).reshape(_TB, 1)
        idx_cols.append(idx)
        onehot = (iota_k == idx).astype(jnp.bfloat16)
        cb_hi = cb.astype(jnp.bfloat16)
        rem1 = cb - cb_hi.astype(jnp.float32)
        cb_mid = rem1.astype(jnp.bfloat16)
        cb_lo = (rem1 - cb_mid.astype(jnp.float32)).astype(jnp.bfloat16)
        q_step = jnp.dot(onehot, cb_hi, preferred_element_type=jnp.float32)
        q_step = q_step + jnp.dot(onehot, cb_mid,
                                  preferred_element_type=jnp.float32)
        q_step = q_step + jnp.dot(onehot, cb_lo,
                                  preferred_element_type=jnp.float32)
        residual = residual - q_step
        quantized = quantized + q_step
        loss = loss + jnp.sum(residual * residual)
    q_ref[...] = quantized
    idx_ref[...] = jnp.concatenate(idx_cols, axis=1)

    @pl.when(step == 0)
    def _init():
        loss_ref[...] = jnp.zeros((1, 1), jnp.float32)

    loss_ref[...] += (loss * _LOSS_SCALE).reshape(1, 1)


@jax.jit
def kernel(x, codebooks):
    xf = x.reshape(_N, _D)
    quantized, indices, loss = pl.pallas_call(
        _rvq_block,
        grid=(_GRID,),
        in_specs=[
            pl.BlockSpec((_TB, _D), lambda i: (i, 0)),
            pl.BlockSpec((_Q, _K, _D), lambda i: (0, 0, 0)),
        ],
        out_specs=[
            pl.BlockSpec((_TB, _D), lambda i: (i, 0)),
            pl.BlockSpec((_TB, _Q), lambda i: (i, 0)),
            pl.BlockSpec((1, 1), lambda i: (0, 0)),
        ],
        out_shape=[
            jax.ShapeDtypeStruct((_N, _D), jnp.float32),
            jax.ShapeDtypeStruct((_N, _Q), jnp.int32),
            jax.ShapeDtypeStruct((1, 1), jnp.float32),
        ],
        compiler_params=pltpu.CompilerParams(
            dimension_semantics=("arbitrary",),
        ),
    )(xf, codebooks)
    return (quantized.reshape(_B, _S, _D),
            indices.reshape(_B, _S, _Q),
            loss.reshape(()))
